# initial kernel scaffold (unmeasured)
import jax
import jax.numpy as jnp
from jax import lax
from jax.experimental import pallas as pl
from jax.experimental.pallas import tpu as pltpu


def kernel(
    x,
):
    def body(*refs):
        pass

    out_shape = jax.ShapeDtypeStruct(..., jnp.float32)
    return pl.pallas_call(body, out_shape=out_shape)(...)


# baseline (device time: 20291 ns/iter reference)
import functools

import jax
import jax.numpy as jnp
from jax import lax
from jax.experimental import pallas as pl
from jax.experimental.pallas import tpu as pltpu

N_DEV = 16


def kernel(x):
    m, n = x.shape

    def body(x_ref, out_ref, comm_ref, send_sems, recv_sems):
        my = lax.axis_index("i")

        comm_ref[my] = jnp.max(x_ref[...], axis=0, keepdims=True)

        barrier = pltpu.get_barrier_semaphore()
        for d in range(1, N_DEV):
            peer = lax.rem(my + d, N_DEV)
            pl.semaphore_signal(
                barrier, inc=1,
                device_id=(peer,), device_id_type=pl.DeviceIdType.MESH,
            )
        pl.semaphore_wait(barrier, N_DEV - 1)

        for d in range(1, N_DEV):
            dst = lax.rem(my + d, N_DEV)
            rdma = pltpu.make_async_remote_copy(
                src_ref=comm_ref.at[my],
                dst_ref=comm_ref.at[my],
                send_sem=send_sems.at[d - 1],
                recv_sem=recv_sems.at[my],
                device_id=(dst,),
                device_id_type=pl.DeviceIdType.MESH,
            )
            rdma.start()

        for d in range(1, N_DEV):
            src = lax.rem(my + d, N_DEV)
            recv = pltpu.make_async_remote_copy(
                src_ref=comm_ref.at[src],
                dst_ref=comm_ref.at[src],
                send_sem=send_sems.at[d - 1],
                recv_sem=recv_sems.at[src],
                device_id=(src,),
                device_id_type=pl.DeviceIdType.MESH,
            )
            recv.wait_recv()

        for d in range(1, N_DEV):
            dst = lax.rem(my + d, N_DEV)
            snd = pltpu.make_async_remote_copy(
                src_ref=comm_ref.at[my],
                dst_ref=comm_ref.at[my],
                send_sem=send_sems.at[d - 1],
                recv_sem=recv_sems.at[my],
                device_id=(dst,),
                device_id_type=pl.DeviceIdType.MESH,
            )
            snd.wait_send()

        out_ref[...] = jnp.max(comm_ref[...], axis=0)

        @functools.partial(pl.run_scoped, sem=pltpu.SemaphoreType.REGULAR)
        def _(sem):
            for d in range(1, N_DEV):
                peer = lax.rem(my + d, N_DEV)
                pl.semaphore_signal(
                    sem, inc=1,
                    device_id=(peer,), device_id_type=pl.DeviceIdType.MESH,
                )
            pl.semaphore_wait(sem, N_DEV - 1)

    return pl.pallas_call(
        body,
        out_shape=jax.ShapeDtypeStruct((1, n), jnp.float32),
        in_specs=[pl.BlockSpec(memory_space=pltpu.VMEM)],
        out_specs=pl.BlockSpec(memory_space=pltpu.VMEM),
        scratch_shapes=[
            pltpu.VMEM((N_DEV, 1, n), jnp.float32),
            pltpu.SemaphoreType.DMA((N_DEV,)),
            pltpu.SemaphoreType.DMA((N_DEV,)),
        ],
        compiler_params=pltpu.CompilerParams(collective_id=0),
    )(x)


# device time: 19480 ns/iter; 1.0416x vs baseline; 1.0416x over previous
import functools

import jax
import jax.numpy as jnp
from jax import lax
from jax.experimental import pallas as pl
from jax.experimental.pallas import tpu as pltpu

N_DEV = 16
GRID = 8


def kernel(x):
    m, n = x.shape
    bm = m // GRID

    def body(x_ref, out_ref, comm_ref, send_sems, recv_sems):
        k = pl.program_id(0)
        my = lax.axis_index("i")
        barrier = pltpu.get_barrier_semaphore()

        @pl.when(k == 0)
        def _():
            for d in range(1, N_DEV):
                peer = lax.rem(my + d, N_DEV)
                pl.semaphore_signal(
                    barrier, inc=1,
                    device_id=(peer,), device_id_type=pl.DeviceIdType.MESH,
                )

        block_max = jnp.max(x_ref[...], axis=0, keepdims=True)

        @pl.when(k == 0)
        def _():
            comm_ref[my] = block_max

        @pl.when(k > 0)
        def _():
            comm_ref[my] = jnp.maximum(comm_ref[my], block_max)

        @pl.when(k == GRID - 1)
        def _():
            pl.semaphore_wait(barrier, N_DEV - 1)

            for d in range(1, N_DEV):
                dst = lax.rem(my + d, N_DEV)
                rdma = pltpu.make_async_remote_copy(
                    src_ref=comm_ref.at[my],
                    dst_ref=comm_ref.at[my],
                    send_sem=send_sems.at[d - 1],
                    recv_sem=recv_sems.at[my],
                    device_id=(dst,),
                    device_id_type=pl.DeviceIdType.MESH,
                )
                rdma.start()

            for d in range(1, N_DEV):
                src = lax.rem(my + d, N_DEV)
                recv = pltpu.make_async_remote_copy(
                    src_ref=comm_ref.at[src],
                    dst_ref=comm_ref.at[src],
                    send_sem=send_sems.at[d - 1],
                    recv_sem=recv_sems.at[src],
                    device_id=(src,),
                    device_id_type=pl.DeviceIdType.MESH,
                )
                recv.wait_recv()

            for d in range(1, N_DEV):
                dst = lax.rem(my + d, N_DEV)
                snd = pltpu.make_async_remote_copy(
                    src_ref=comm_ref.at[my],
                    dst_ref=comm_ref.at[my],
                    send_sem=send_sems.at[d - 1],
                    recv_sem=recv_sems.at[my],
                    device_id=(dst,),
                    device_id_type=pl.DeviceIdType.MESH,
                )
                snd.wait_send()

            out_ref[...] = jnp.max(comm_ref[...], axis=0)

            @functools.partial(
                pl.run_scoped, sem=pltpu.SemaphoreType.REGULAR
            )
            def _(sem):
                for d in range(1, N_DEV):
                    peer = lax.rem(my + d, N_DEV)
                    pl.semaphore_signal(
                        sem, inc=1,
                        device_id=(peer,),
                        device_id_type=pl.DeviceIdType.MESH,
                    )
                pl.semaphore_wait(sem, N_DEV - 1)

    return pl.pallas_call(
        body,
        grid=(GRID,),
        out_shape=jax.ShapeDtypeStruct((1, n), jnp.float32),
        in_specs=[pl.BlockSpec((bm, n), lambda k: (k, 0))],
        out_specs=pl.BlockSpec((1, n), lambda k: (0, 0)),
        scratch_shapes=[
            pltpu.VMEM((N_DEV, 1, n), jnp.float32),
            pltpu.SemaphoreType.DMA((N_DEV,)),
            pltpu.SemaphoreType.DMA((N_DEV,)),
        ],
        compiler_params=pltpu.CompilerParams(collective_id=0),
    )(x)
